# baseline (speedup 1.0000x reference)
import jax
import jax.numpy as jnp
from jax.experimental import pallas as pl
from jax.experimental.pallas import tpu as pltpu

_DIM = 128
_TOTAL = 32768
_NLEV = 15
_ILP = 16


_CMAX = 128


def _chunk_size(t):
    max_p = (_TOTAL >> (t + 1)) + 2
    c = 8
    while c < max_p and c < _CMAX:
        c *= 2
    return c


_MAXCH = _TOTAL // (2 * _CMAX) + 32


def _tree_kernel(limits_ref, args_ref, w1t_ref, b1_ref, w2t_ref, b2_ref,
                 out_ref, bufa_ref, bufb_ref,
                 cs_ref, ln_ref, tin_ref, tout_ref, lsrc_ref, ldst_ref):
    nseg = limits_ref.shape[0] - 1

    def mlp(x2):
        h = jnp.dot(x2, w1t_ref[...], preferred_element_type=jnp.float32)
        h = jnp.maximum(h + b1_ref[...], 0.0)
        y = jnp.dot(h, w2t_ref[...], preferred_element_type=jnp.float32)
        return y + b2_ref[...]

    for t in range(_NLEV):
        C = _chunk_size(t)
        in_ref = args_ref if t == 0 else (bufb_ref if t % 2 == 0 else bufa_ref)
        dst_buf = bufa_ref if t % 2 == 0 else bufb_ref

        def build_seg(s, carry, t=t, C=C):
            ci, li, dcum = carry
            if t == 0:
                src = limits_ref[s]
                length = limits_ref[s + 1] - src
            else:
                src = cs_ref[s]
                length = ln_ref[s]
            p = length // 2
            odd = length - 2 * p
            nch = (p + C - 1) // C

            def put(j, ci):
                if t == 0:
                    offp = jnp.maximum(jnp.minimum(C * j, p - C),
                                       -(src >> 1))
                else:
                    offp = C * j
                tin_ref[ci] = src + 2 * offp
                tout_ref[ci] = dcum + odd + offp
                return ci + 1

            ci = jax.lax.fori_loop(0, nch, put, ci)

            @pl.when(odd == 1)
            def _():
                lsrc_ref[li] = src + 2 * p
                ldst_ref[li] = dcum

            li = li + odd
            cs_ref[s] = dcum
            ln_ref[s] = p + odd
            return ci, li, dcum + p + odd + C

        nch_all, nleft, _ = jax.lax.fori_loop(
            0, nseg, build_seg, (jnp.int32(0), jnp.int32(0), jnp.int32(0)))

        def pad_dup(k, _):
            @pl.when(k >= nch_all)
            def _():
                tin_ref[k] = tin_ref[nch_all - 1]
                tout_ref[k] = tout_ref[nch_all - 1]
            return 0

        npad = (nch_all + _ILP - 1) // _ILP * _ILP

        @pl.when(nch_all > 0)
        def _():
            jax.lax.fori_loop(nch_all, npad, pad_dup, 0)

        def chunk_group(c, _, C=C, in_ref=in_ref, dst_buf=dst_buf):
            for u in range(_ILP):
                k = _ILP * c + u
                ib = tin_ref[k]
                ob = tout_ref[k]
                x = in_ref[pl.ds(ib, 2 * C), :]
                dst_buf[pl.ds(ob, C), :] = mlp(x.reshape(C, 2 * _DIM))
            return 0

        jax.lax.fori_loop(0, npad // _ILP, chunk_group, 0)

        def left_copy(k, _, in_ref=in_ref, dst_buf=dst_buf):
            dst_buf[pl.ds(ldst_ref[k], 1), :] = in_ref[pl.ds(lsrc_ref[k], 1), :]
            return 0

        jax.lax.fori_loop(0, nleft, left_copy, 0)

    def out_seg(s, _):
        out_ref[pl.ds(s, 1), :] = bufa_ref[pl.ds(cs_ref[s], 1), :]
        return 0

    jax.lax.fori_loop(0, nseg, out_seg, 0)


def kernel(args, limits, W1, b1, W2, b2):
    total, dim = args.shape
    nseg = limits.shape[0] - 1
    c0 = _chunk_size(0)
    buf_rows = (total + nseg) // 2 + (nseg + 3) * c0
    buf_rows = (buf_rows + 7) // 8 * 8

    out = pl.pallas_call(
        _tree_kernel,
        out_shape=jax.ShapeDtypeStruct((nseg, dim), jnp.float32),
        in_specs=[
            pl.BlockSpec(memory_space=pltpu.SMEM),
            pl.BlockSpec(memory_space=pltpu.VMEM),
            pl.BlockSpec(memory_space=pltpu.VMEM),
            pl.BlockSpec(memory_space=pltpu.VMEM),
            pl.BlockSpec(memory_space=pltpu.VMEM),
            pl.BlockSpec(memory_space=pltpu.VMEM),
        ],
        out_specs=pl.BlockSpec(memory_space=pltpu.VMEM),
        scratch_shapes=[
            pltpu.VMEM((buf_rows, dim), jnp.float32),
            pltpu.VMEM((buf_rows, dim), jnp.float32),
            pltpu.SMEM((nseg,), jnp.int32),
            pltpu.SMEM((nseg,), jnp.int32),
            pltpu.SMEM((_MAXCH,), jnp.int32),
            pltpu.SMEM((_MAXCH,), jnp.int32),
            pltpu.SMEM((nseg,), jnp.int32),
            pltpu.SMEM((nseg,), jnp.int32),
        ],
    )(
        limits.astype(jnp.int32),
        args,
        W1.T,
        b1.reshape(1, -1),
        W2.T,
        b2.reshape(1, -1),
    )
    return out


# speedup vs baseline: 1.0268x; 1.0268x over previous
import jax
import jax.numpy as jnp
from jax.experimental import pallas as pl
from jax.experimental.pallas import tpu as pltpu

_DIM = 128
_TOTAL = 32768
_NLEV = 15
_ILP = 16


_CMAX = 256


def _chunk_size(t):
    max_p = (_TOTAL >> (t + 1)) + 2
    c = 8
    while c < max_p and c < _CMAX:
        c *= 2
    return c


_MAXCH = _TOTAL // (2 * _CMAX) + 32


def _tree_kernel(limits_ref, args_ref, w1t_ref, b1_ref, w2t_ref, b2_ref,
                 out_ref, bufa_ref, bufb_ref,
                 cs_ref, ln_ref, tin_ref, tout_ref, lsrc_ref, ldst_ref):
    nseg = limits_ref.shape[0] - 1

    def mlp(x2):
        h = jnp.dot(x2, w1t_ref[...], preferred_element_type=jnp.float32)
        h = jnp.maximum(h + b1_ref[...], 0.0)
        y = jnp.dot(h, w2t_ref[...], preferred_element_type=jnp.float32)
        return y + b2_ref[...]

    for t in range(_NLEV):
        C = _chunk_size(t)
        in_ref = args_ref if t == 0 else (bufb_ref if t % 2 == 0 else bufa_ref)
        dst_buf = bufa_ref if t % 2 == 0 else bufb_ref

        def build_seg(s, carry, t=t, C=C):
            ci, li, dcum = carry
            if t == 0:
                src = limits_ref[s]
                length = limits_ref[s + 1] - src
            else:
                src = cs_ref[s]
                length = ln_ref[s]
            p = length // 2
            odd = length - 2 * p
            nch = (p + C - 1) // C

            def put(j, ci):
                if t == 0:
                    offp = jnp.maximum(jnp.minimum(C * j, p - C),
                                       -(src >> 1))
                else:
                    offp = C * j
                tin_ref[ci] = src + 2 * offp
                tout_ref[ci] = dcum + odd + offp
                return ci + 1

            ci = jax.lax.fori_loop(0, nch, put, ci)

            @pl.when(odd == 1)
            def _():
                lsrc_ref[li] = src + 2 * p
                ldst_ref[li] = dcum

            li = li + odd
            cs_ref[s] = dcum
            ln_ref[s] = p + odd
            return ci, li, dcum + p + odd + C

        nch_all, nleft, _ = jax.lax.fori_loop(
            0, nseg, build_seg, (jnp.int32(0), jnp.int32(0), jnp.int32(0)))

        def pad_dup(k, _):
            @pl.when(k >= nch_all)
            def _():
                tin_ref[k] = tin_ref[nch_all - 1]
                tout_ref[k] = tout_ref[nch_all - 1]
            return 0

        npad = (nch_all + _ILP - 1) // _ILP * _ILP

        @pl.when(nch_all > 0)
        def _():
            jax.lax.fori_loop(nch_all, npad, pad_dup, 0)

        def chunk_group(c, _, C=C, in_ref=in_ref, dst_buf=dst_buf):
            for u in range(_ILP):
                k = _ILP * c + u
                ib = tin_ref[k]
                ob = tout_ref[k]
                x = in_ref[pl.ds(ib, 2 * C), :]
                dst_buf[pl.ds(ob, C), :] = mlp(x.reshape(C, 2 * _DIM))
            return 0

        jax.lax.fori_loop(0, npad // _ILP, chunk_group, 0)

        def left_copy(k, _, in_ref=in_ref, dst_buf=dst_buf):
            dst_buf[pl.ds(ldst_ref[k], 1), :] = in_ref[pl.ds(lsrc_ref[k], 1), :]
            return 0

        jax.lax.fori_loop(0, nleft, left_copy, 0)

    def out_seg(s, _):
        out_ref[pl.ds(s, 1), :] = bufa_ref[pl.ds(cs_ref[s], 1), :]
        return 0

    jax.lax.fori_loop(0, nseg, out_seg, 0)


def kernel(args, limits, W1, b1, W2, b2):
    total, dim = args.shape
    nseg = limits.shape[0] - 1
    c0 = _chunk_size(0)
    buf_rows = (total + nseg) // 2 + (nseg + 3) * c0
    buf_rows = (buf_rows + 7) // 8 * 8

    out = pl.pallas_call(
        _tree_kernel,
        out_shape=jax.ShapeDtypeStruct((nseg, dim), jnp.float32),
        in_specs=[
            pl.BlockSpec(memory_space=pltpu.SMEM),
            pl.BlockSpec(memory_space=pltpu.VMEM),
            pl.BlockSpec(memory_space=pltpu.VMEM),
            pl.BlockSpec(memory_space=pltpu.VMEM),
            pl.BlockSpec(memory_space=pltpu.VMEM),
            pl.BlockSpec(memory_space=pltpu.VMEM),
        ],
        out_specs=pl.BlockSpec(memory_space=pltpu.VMEM),
        scratch_shapes=[
            pltpu.VMEM((buf_rows, dim), jnp.float32),
            pltpu.VMEM((buf_rows, dim), jnp.float32),
            pltpu.SMEM((nseg,), jnp.int32),
            pltpu.SMEM((nseg,), jnp.int32),
            pltpu.SMEM((_MAXCH,), jnp.int32),
            pltpu.SMEM((_MAXCH,), jnp.int32),
            pltpu.SMEM((nseg,), jnp.int32),
            pltpu.SMEM((nseg,), jnp.int32),
        ],
    )(
        limits.astype(jnp.int32),
        args,
        W1.T,
        b1.reshape(1, -1),
        W2.T,
        b2.reshape(1, -1),
    )
    return out
